# baseline (speedup 1.0000x reference)
import functools

import jax
import jax.numpy as jnp
from jax import lax
from jax.experimental import pallas as pl
from jax.experimental.pallas import tpu as pltpu
from jax.experimental.pallas import tpu_sc as plsc

BATCH = 16384
ATOMS = 200
NTAB = 4
LANES = 16

NW = 32
RPW = BATCH // NW
CH = 64
NCH = RPW // CH
CW = CH * ATOMS
NSLC = ATOMS // LANES
NGRP = CH // LANES

_mesh = plsc.VectorSubcoreMesh(core_axis_name="c", subcore_axis_name="s")


@functools.partial(
    pl.kernel,
    mesh=_mesh,
    out_type=jax.ShapeDtypeStruct((BATCH,), jnp.float32),
    compiler_params=pltpu.CompilerParams(needs_layout_passes=False),
    scratch_types=[
        pltpu.VMEM((128,), jnp.float32),
        pltpu.VMEM((RPW,), jnp.float32),
        pltpu.VMEM((RPW,), jnp.float32),
        pltpu.VMEM((CW + LANES,), jnp.int32),
        pltpu.VMEM((CW + LANES,), jnp.int32),
        pltpu.VMEM((CH * LANES,), jnp.float32),
        pltpu.SemaphoreType.DMA,
        pltpu.SemaphoreType.DMA,
    ],
)
def _sc_kernel(en_hbm, sp_hbm, se_hbm, out_hbm,
               se_v, en_v, out_v, sp_v0, sp_v1, stage_v, sem0, sem1):
    wid = lax.axis_index("s") * 2 + lax.axis_index("c")
    base = wid * RPW

    pltpu.sync_copy(se_hbm, se_v.at[pl.ds(0, NTAB)])
    pltpu.sync_copy(en_hbm.at[pl.ds(base, RPW)], en_v)

    bufs = (sp_v0, sp_v1)
    sems = (sem0, sem1)

    def start(c):
        return pltpu.async_copy(
            sp_hbm.at[pl.ds((base + c * CH) * ATOMS, CW)],
            bufs[c % 2].at[pl.ds(0, CW)],
            sems[c % 2],
        )

    lane = lax.iota(jnp.int32, LANES)
    tail_mask = lane < (ATOMS - NSLC * LANES)
    zeros = jnp.zeros((LANES,), jnp.float32)

    pending = start(0)
    for c in range(NCH):
        nxt = start(c + 1) if c + 1 < NCH else None
        pending.wait()
        buf = bufs[c % 2]

        def mol_body(m, _):
            off = m * ATOMS
            acc = zeros
            for k in range(NSLC):
                sp = buf[pl.ds(off + k * LANES, LANES)]
                acc = acc + plsc.load_gather(se_v, [sp])
            spt = buf[pl.ds(off + NSLC * LANES, LANES)] & 3
            g = plsc.load_gather(se_v, [spt])
            acc = acc + jnp.where(tail_mask, g, zeros)
            stage_v[pl.ds(m * LANES, LANES)] = acc
            return 0

        lax.fori_loop(0, CH, mol_body, 0)

        def grp_body(g, _):
            row0 = g * LANES
            acc2 = zeros
            idx0 = (row0 + lane) * LANES
            for p in range(LANES):
                acc2 = acc2 + plsc.load_gather(stage_v, [idx0 + p])
            en = en_v[pl.ds(c * CH + row0, LANES)]
            out_v[pl.ds(c * CH + row0, LANES)] = en - acc2
            return 0

        lax.fori_loop(0, NGRP, grp_body, 0)
        pending = nxt

    pltpu.sync_copy(out_v, out_hbm.at[pl.ds(base, RPW)])


def kernel(energies, species, self_energies):
    sp_flat = species.reshape(BATCH * ATOMS)
    return _sc_kernel(energies, sp_flat, self_energies)


# speedup vs baseline: 1.5213x; 1.5213x over previous
import functools

import jax
import jax.numpy as jnp
from jax import lax
from jax.experimental import pallas as pl
from jax.experimental.pallas import tpu as pltpu
from jax.experimental.pallas import tpu_sc as plsc

BATCH = 16384
ATOMS = 200
NTAB = 4
LANES = 16

NW = 32
RPW = BATCH // NW
CH = 64
NCH = RPW // CH
NSLC = ATOMS // LANES
NGRP = CH // LANES
TAIL = ATOMS - LANES

_mesh = plsc.VectorSubcoreMesh(core_axis_name="c", subcore_axis_name="s")


@functools.partial(
    pl.kernel,
    mesh=_mesh,
    out_type=jax.ShapeDtypeStruct((BATCH,), jnp.float32),
    compiler_params=pltpu.CompilerParams(needs_layout_passes=False),
    scratch_types=[
        pltpu.VMEM((128,), jnp.float32),
        pltpu.VMEM((RPW,), jnp.float32),
        pltpu.VMEM((RPW,), jnp.float32),
        pltpu.VMEM((CH, ATOMS), jnp.int32),
        pltpu.VMEM((CH, ATOMS), jnp.int32),
        pltpu.VMEM((CH * LANES,), jnp.float32),
        pltpu.SemaphoreType.DMA,
        pltpu.SemaphoreType.DMA,
    ],
)
def _sc_kernel(en_hbm, sp_hbm, se_hbm, out_hbm,
               se_v, en_v, out_v, sp_v0, sp_v1, stage_v, sem0, sem1):
    wid = lax.axis_index("s") * 2 + lax.axis_index("c")
    base = wid * RPW

    pltpu.sync_copy(se_hbm, se_v.at[pl.ds(0, NTAB)])
    pltpu.sync_copy(en_hbm.at[pl.ds(base, RPW)], en_v)

    bufs = (sp_v0, sp_v1)
    sems = (sem0, sem1)

    def start(c):
        return pltpu.async_copy(
            sp_hbm.at[pl.ds(base + c * CH, CH), :],
            bufs[c % 2],
            sems[c % 2],
        )

    lane = lax.iota(jnp.int32, LANES)
    tail_mask = lane >= (LANES - (ATOMS - NSLC * LANES))
    zeros = jnp.zeros((LANES,), jnp.float32)

    pending = start(0)
    for c in range(NCH):
        nxt = start(c + 1) if c + 1 < NCH else None
        pending.wait()
        buf = bufs[c % 2]

        def mol_body(m, _):
            g = [plsc.load_gather(se_v, [buf[m, pl.ds(k * LANES, LANES)]])
                 for k in range(NSLC)]
            gt = plsc.load_gather(se_v, [buf[m, pl.ds(TAIL, LANES)]])
            g.append(jnp.where(tail_mask, gt, zeros))
            while len(g) > 1:
                g = [a + b for a, b in zip(g[0::2], g[1::2])] + (
                    [g[-1]] if len(g) % 2 else [])
            stage_v[pl.ds(m * LANES, LANES)] = g[0]
            return 0

        lax.fori_loop(0, CH, mol_body, 0)

        def grp_body(gi, _):
            row0 = gi * LANES
            idx0 = (row0 + lane) * LANES
            acc = zeros
            for p in range(LANES):
                acc = acc + plsc.load_gather(stage_v, [idx0 + p])
            en = en_v[pl.ds(c * CH + row0, LANES)]
            out_v[pl.ds(c * CH + row0, LANES)] = en - acc
            return 0

        lax.fori_loop(0, NGRP, grp_body, 0)
        pending = nxt

    pltpu.sync_copy(out_v, out_hbm.at[pl.ds(base, RPW)])


def kernel(energies, species, self_energies):
    return _sc_kernel(energies, species, self_energies)


# speedup vs baseline: 1.5316x; 1.0068x over previous
import functools

import jax
import jax.numpy as jnp
from jax import lax
from jax.experimental import pallas as pl
from jax.experimental.pallas import tpu as pltpu
from jax.experimental.pallas import tpu_sc as plsc

BATCH = 16384
ATOMS = 200
NTAB = 4
LANES = 16

NW = 32
RPW = BATCH // NW
CH = 64
NCH = RPW // CH
NSLC = ATOMS // LANES
NGRP = CH // LANES
TAIL = ATOMS - LANES

_mesh = plsc.VectorSubcoreMesh(core_axis_name="c", subcore_axis_name="s")


@functools.partial(
    pl.kernel,
    mesh=_mesh,
    out_type=jax.ShapeDtypeStruct((BATCH,), jnp.float32),
    compiler_params=pltpu.CompilerParams(
        needs_layout_passes=False, use_tc_tiling_on_sc=True),
    scratch_types=[
        pltpu.VMEM((128,), jnp.float32),
        pltpu.VMEM((RPW,), jnp.float32),
        pltpu.VMEM((RPW,), jnp.float32),
        pltpu.VMEM((CH, ATOMS), jnp.int32),
        pltpu.VMEM((CH, ATOMS), jnp.int32),
        pltpu.VMEM((CH * LANES,), jnp.float32),
        pltpu.SemaphoreType.DMA,
        pltpu.SemaphoreType.DMA,
    ],
)
def _sc_kernel(en_hbm, sp_hbm, se_hbm, out_hbm,
               se_v, en_v, out_v, sp_v0, sp_v1, stage_v, sem0, sem1):
    wid = lax.axis_index("s") * 2 + lax.axis_index("c")
    base = wid * RPW

    pltpu.sync_copy(se_hbm, se_v.at[pl.ds(0, NTAB)])
    pltpu.sync_copy(en_hbm.at[pl.ds(base, RPW)], en_v)

    bufs = (sp_v0, sp_v1)
    sems = (sem0, sem1)

    def start(c):
        return pltpu.async_copy(
            sp_hbm.at[pl.ds(base + c * CH, CH), :],
            bufs[c % 2],
            sems[c % 2],
        )

    lane = lax.iota(jnp.int32, LANES)
    tail_mask = lane >= (LANES - (ATOMS - NSLC * LANES))
    zeros = jnp.zeros((LANES,), jnp.float32)

    pending = start(0)
    for c in range(NCH):
        nxt = start(c + 1) if c + 1 < NCH else None
        pending.wait()
        buf = bufs[c % 2]

        def mol_body(m, _):
            g = [plsc.load_gather(se_v, [buf[m, pl.ds(k * LANES, LANES)]])
                 for k in range(NSLC)]
            gt = plsc.load_gather(se_v, [buf[m, pl.ds(TAIL, LANES)]])
            g.append(jnp.where(tail_mask, gt, zeros))
            while len(g) > 1:
                g = [a + b for a, b in zip(g[0::2], g[1::2])] + (
                    [g[-1]] if len(g) % 2 else [])
            stage_v[pl.ds(m * LANES, LANES)] = g[0]
            return 0

        lax.fori_loop(0, CH, mol_body, 0)

        def grp_body(gi, _):
            row0 = gi * LANES
            idx0 = (row0 + lane) * LANES
            acc = zeros
            for p in range(LANES):
                acc = acc + plsc.load_gather(stage_v, [idx0 + p])
            en = en_v[pl.ds(c * CH + row0, LANES)]
            out_v[pl.ds(c * CH + row0, LANES)] = en - acc
            return 0

        lax.fori_loop(0, NGRP, grp_body, 0)
        pending = nxt

    pltpu.sync_copy(out_v, out_hbm.at[pl.ds(base, RPW)])


def kernel(energies, species, self_energies):
    return _sc_kernel(energies, species, self_energies)
